# Initial kernel scaffold; baseline (speedup 1.0000x reference)
import jax
import jax.numpy as jnp
from jax.experimental import pallas as pl


def kernel(h, edge_index, W, b):
    raise NotImplementedError("write your pallas kernel here")


# speedup vs baseline: 3.3225x; 3.3225x over previous
import functools

import jax
import jax.numpy as jnp
from jax import lax
from jax.experimental import pallas as pl
from jax.experimental.pallas import tpu as pltpu
from jax.experimental.pallas import tpu_sc as plsc

N_NODES = 10000
D = 128
NC = 2
NS = 16
CHUNK = 128
GRP = 8
N_PAD = 10112
BAND = N_PAD // NS
DEGW = 16


def _sc_aggregate(h, src, dst, groups_per_tile):
    mesh = plsc.VectorSubcoreMesh(core_axis_name="c", subcore_axis_name="s")

    @functools.partial(
        pl.kernel,
        out_type=[
            jax.ShapeDtypeStruct((NC, N_PAD, D), jnp.float32),
            jax.ShapeDtypeStruct((NC, N_PAD, DEGW), jnp.float32),
        ],
        mesh=mesh,
        compiler_params=pltpu.CompilerParams(use_tc_tiling_on_sc=False),
        scratch_types=[
            pltpu.VMEM((GRP, CHUNK), jnp.int32),
            pltpu.VMEM((GRP, CHUNK), jnp.int32),
            pltpu.VMEM((2, CHUNK, D), jnp.float32),
            pltpu.VMEM((CHUNK, DEGW), jnp.float32),
            pltpu.VMEM((8, D), jnp.float32),
            pltpu.VMEM((8, DEGW), jnp.float32),
            pltpu.VMEM_SHARED((N_PAD, D), jnp.float32),
            pltpu.VMEM_SHARED((N_PAD, DEGW), jnp.float32),
            pltpu.SemaphoreType.DMA,
        ],
    )
    def agg_kernel(h_hbm, src_hbm, dst_hbm, agg_out, deg_out,
                   src_v, dst_v, rows_v, ones_v, zagg_v, zdeg_v,
                   agg_sh, deg_sh, sem):
        cid = lax.axis_index("c")
        sid = lax.axis_index("s")

        ones16 = jnp.ones((16,), jnp.float32)
        zeros16 = jnp.zeros((16,), jnp.float32)

        def fill_ones(i, _):
            ones_v[i] = ones16
            return 0

        lax.fori_loop(0, CHUNK, fill_ones, 0)

        def fill_z(i, _):
            for j in range(D // 16):
                zagg_v[i, pl.ds(j * 16, 16)] = zeros16
            zdeg_v[i] = zeros16
            return 0

        lax.fori_loop(0, 8, fill_z, 0)

        base = sid * BAND

        def zero_band(t, _):
            pltpu.sync_copy(zagg_v, agg_sh.at[pl.ds(base + t * 8, 8)])
            pltpu.sync_copy(zdeg_v, deg_sh.at[pl.ds(base + t * 8, 8)])
            return 0

        lax.fori_loop(0, BAND // 8, zero_band, 0)
        plsc.subcore_barrier()

        def group(g, _):
            pltpu.sync_copy(src_hbm.at[cid, sid, pl.ds(g * GRP, GRP)], src_v)
            pltpu.sync_copy(dst_hbm.at[cid, sid, pl.ds(g * GRP, GRP)], dst_v)
            for j in range(GRP):
                buf = rows_v.at[j % 2]
                pltpu.async_copy(h_hbm.at[src_v.at[j]], buf, sem).wait()
                pltpu.sync_copy(buf, agg_sh.at[dst_v.at[j]], add=True)
                pltpu.sync_copy(ones_v, deg_sh.at[dst_v.at[j]], add=True)
            return 0

        lax.fori_loop(0, groups_per_tile, group, 0)
        plsc.subcore_barrier()

        pltpu.sync_copy(agg_sh.at[pl.ds(base, BAND)],
                        agg_out.at[cid, pl.ds(base, BAND)])
        pltpu.sync_copy(deg_sh.at[pl.ds(base, BAND)],
                        deg_out.at[cid, pl.ds(base, BAND)])

    return agg_kernel(h, src, dst)


def _tc_body(h_ref, a0_ref, a1_ref, d0_ref, d1_ref, w_ref, b_ref, o_ref):
    deg = d0_ref[:, 0:1] + d1_ref[:, 0:1]
    agg = a0_ref[...] + a1_ref[...]
    mean = agg / jnp.maximum(deg, 1.0)
    h_new = jnp.where(deg > 0.0, mean, h_ref[...])
    acc = jnp.dot(h_new, w_ref[...], preferred_element_type=jnp.float32)
    o_ref[...] = jnp.maximum(acc + b_ref[...], 0.0)


def _tc_update(h, a0, a1, d0, d1, W, b):
    R = 2000
    grid = (N_NODES // R,)
    return pl.pallas_call(
        _tc_body,
        grid=grid,
        in_specs=[
            pl.BlockSpec((R, D), lambda i: (i, 0)),
            pl.BlockSpec((R, D), lambda i: (i, 0)),
            pl.BlockSpec((R, D), lambda i: (i, 0)),
            pl.BlockSpec((R, DEGW), lambda i: (i, 0)),
            pl.BlockSpec((R, DEGW), lambda i: (i, 0)),
            pl.BlockSpec((D, D), lambda i: (0, 0)),
            pl.BlockSpec((1, D), lambda i: (0, 0)),
        ],
        out_specs=pl.BlockSpec((R, D), lambda i: (i, 0)),
        out_shape=jax.ShapeDtypeStruct((N_NODES, D), jnp.float32),
    )(h, a0, a1, d0, d1, W, b)


def kernel(h, edge_index, W, b):
    src = edge_index[0].astype(jnp.int32)
    dst = edge_index[1].astype(jnp.int32)
    E = src.shape[0]
    lane = NC * NS * GRP * CHUNK
    groups_per_tile = -(-E // lane)
    e_pad = lane * groups_per_tile
    if e_pad != E:
        src = jnp.concatenate(
            [src, jnp.zeros((e_pad - E,), jnp.int32)])
        dst = jnp.concatenate(
            [dst, jnp.full((e_pad - E,), N_NODES, jnp.int32)])
    src = src.reshape(NC, NS, groups_per_tile * GRP, CHUNK)
    dst = dst.reshape(NC, NS, groups_per_tile * GRP, CHUNK)

    agg_p, deg_p = _sc_aggregate(h, src, dst, groups_per_tile)

    return _tc_update(
        h,
        agg_p[0, :N_NODES], agg_p[1, :N_NODES],
        deg_p[0, :N_NODES], deg_p[1, :N_NODES],
        W, b.reshape(1, D),
    )


# speedup vs baseline: 3.6010x; 1.0838x over previous
import functools

import jax
import jax.numpy as jnp
from jax import lax
from jax.experimental import pallas as pl
from jax.experimental.pallas import tpu as pltpu
from jax.experimental.pallas import tpu_sc as plsc

N_NODES = 10000
D = 128
NC = 2
NS = 16
CHUNK = 128
GRP = 8
N_PAD = 10112
BAND = N_PAD // NS
DEGW = 16


def _sc_aggregate(h, src, dst, groups_per_tile):
    mesh = plsc.VectorSubcoreMesh(core_axis_name="c", subcore_axis_name="s")

    @functools.partial(
        pl.kernel,
        out_type=[
            jax.ShapeDtypeStruct((NC, N_PAD, D), jnp.float32),
            jax.ShapeDtypeStruct((NC, N_PAD, DEGW), jnp.float32),
        ],
        mesh=mesh,
        compiler_params=pltpu.CompilerParams(use_tc_tiling_on_sc=False),
        scratch_types=[
            pltpu.VMEM((2, GRP, CHUNK), jnp.int32),
            pltpu.VMEM((2, GRP, CHUNK), jnp.int32),
            pltpu.VMEM((2, CHUNK, D), jnp.float32),
            pltpu.VMEM((CHUNK, DEGW), jnp.float32),
            pltpu.VMEM((4, D), jnp.float32),
            pltpu.VMEM((4, DEGW), jnp.float32),
            pltpu.VMEM_SHARED((N_PAD, D), jnp.float32),
            pltpu.VMEM_SHARED((N_PAD, DEGW), jnp.float32),
            pltpu.SemaphoreType.DMA,
            pltpu.SemaphoreType.DMA,
            pltpu.SemaphoreType.DMA,
            pltpu.SemaphoreType.DMA,
            pltpu.SemaphoreType.DMA,
            pltpu.SemaphoreType.DMA,
        ],
    )
    def agg_kernel(h_hbm, src_hbm, dst_hbm, agg_out, deg_out,
                   src_v, dst_v, rows_v, ones_v, zagg_v, zdeg_v,
                   agg_sh, deg_sh, isem, gsem0, gsem1, ssem0, ssem1, dsem):
        cid = lax.axis_index("c")
        sid = lax.axis_index("s")
        gsems = (gsem0, gsem1)
        ssems = (ssem0, ssem1)

        ones16 = jnp.ones((16,), jnp.float32)
        zeros16 = jnp.zeros((16,), jnp.float32)

        def fill_ones(i, _):
            ones_v[i] = ones16
            return 0

        lax.fori_loop(0, CHUNK, fill_ones, 0)

        def fill_z(i, _):
            for j in range(D // 16):
                zagg_v[i, pl.ds(j * 16, 16)] = zeros16
            zdeg_v[i] = zeros16
            return 0

        lax.fori_loop(0, 4, fill_z, 0)

        pltpu.async_copy(src_hbm.at[cid, sid, pl.ds(0, GRP)], src_v.at[0], isem)
        pltpu.async_copy(dst_hbm.at[cid, sid, pl.ds(0, GRP)], dst_v.at[0], isem)

        base = sid * BAND

        def zero_band(t, _):
            pltpu.sync_copy(zagg_v, agg_sh.at[pl.ds(base + t * 4, 4)])
            pltpu.sync_copy(zdeg_v, deg_sh.at[pl.ds(base + t * 4, 4)])
            return 0

        lax.fori_loop(0, BAND // 4, zero_band, 0)
        plsc.subcore_barrier()

        def group(g, _):
            cur = lax.rem(g, 2)
            sv = src_v.at[cur]
            dv = dst_v.at[cur]
            pltpu.make_async_copy(
                src_hbm.at[cid, sid, pl.ds(g * GRP, GRP)], sv, isem).wait()
            pltpu.make_async_copy(
                dst_hbm.at[cid, sid, pl.ds(g * GRP, GRP)], dv, isem).wait()

            @pl.when(g + 1 < groups_per_tile)
            def _prefetch():
                nxt = 1 - cur
                pltpu.async_copy(
                    src_hbm.at[cid, sid, pl.ds((g + 1) * GRP, GRP)],
                    src_v.at[nxt], isem)
                pltpu.async_copy(
                    dst_hbm.at[cid, sid, pl.ds((g + 1) * GRP, GRP)],
                    dst_v.at[nxt], isem)

            gd = {}
            sd = {}
            gd[0] = pltpu.async_copy(h_hbm.at[sv.at[0]], rows_v.at[0], gsem0)
            for j in range(GRP):
                b = j % 2
                gd[j].wait()
                if j + 1 < GRP:
                    if j >= 1:
                        sd[j - 1].wait()
                    gd[j + 1] = pltpu.async_copy(
                        h_hbm.at[sv.at[j + 1]], rows_v.at[1 - b], gsems[1 - b])
                sd[j] = pltpu.async_copy(
                    rows_v.at[b], agg_sh.at[dv.at[j]], ssems[b], add=True)
                pltpu.async_copy(ones_v, deg_sh.at[dv.at[j]], dsem, add=True)
            sd[GRP - 2].wait()
            sd[GRP - 1].wait()
            for j in range(GRP):
                pltpu.make_async_copy(
                    ones_v, deg_sh.at[dv.at[j]], dsem).wait()
            return 0

        lax.fori_loop(0, groups_per_tile, group, 0)
        plsc.subcore_barrier()

        pltpu.sync_copy(agg_sh.at[pl.ds(base, BAND)],
                        agg_out.at[cid, pl.ds(base, BAND)])
        pltpu.sync_copy(deg_sh.at[pl.ds(base, BAND)],
                        deg_out.at[cid, pl.ds(base, BAND)])

    return agg_kernel(h, src, dst)


def _tc_body(h_ref, a0_ref, a1_ref, d0_ref, d1_ref, w_ref, b_ref, o_ref):
    deg = d0_ref[:, 0:1] + d1_ref[:, 0:1]
    agg = a0_ref[...] + a1_ref[...]
    mean = agg / jnp.maximum(deg, 1.0)
    h_new = jnp.where(deg > 0.0, mean, h_ref[...])
    acc = jnp.dot(h_new, w_ref[...], preferred_element_type=jnp.float32)
    o_ref[...] = jnp.maximum(acc + b_ref[...], 0.0)


def _tc_update(h, a0, a1, d0, d1, W, b):
    R = 2000
    grid = (N_NODES // R,)
    return pl.pallas_call(
        _tc_body,
        grid=grid,
        in_specs=[
            pl.BlockSpec((R, D), lambda i: (i, 0)),
            pl.BlockSpec((R, D), lambda i: (i, 0)),
            pl.BlockSpec((R, D), lambda i: (i, 0)),
            pl.BlockSpec((R, DEGW), lambda i: (i, 0)),
            pl.BlockSpec((R, DEGW), lambda i: (i, 0)),
            pl.BlockSpec((D, D), lambda i: (0, 0)),
            pl.BlockSpec((1, D), lambda i: (0, 0)),
        ],
        out_specs=pl.BlockSpec((R, D), lambda i: (i, 0)),
        out_shape=jax.ShapeDtypeStruct((N_NODES, D), jnp.float32),
    )(h, a0, a1, d0, d1, W, b)


def kernel(h, edge_index, W, b):
    src = edge_index[0].astype(jnp.int32)
    dst = edge_index[1].astype(jnp.int32)
    E = src.shape[0]
    lane = NC * NS * GRP * CHUNK
    groups_per_tile = -(-E // lane)
    e_pad = lane * groups_per_tile
    if e_pad != E:
        src = jnp.concatenate(
            [src, jnp.zeros((e_pad - E,), jnp.int32)])
        dst = jnp.concatenate(
            [dst, jnp.full((e_pad - E,), N_NODES, jnp.int32)])
    src = src.reshape(NC, NS, groups_per_tile * GRP, CHUNK)
    dst = dst.reshape(NC, NS, groups_per_tile * GRP, CHUNK)

    agg_p, deg_p = _sc_aggregate(h, src, dst, groups_per_tile)

    return _tc_update(
        h,
        agg_p[0, :N_NODES], agg_p[1, :N_NODES],
        deg_p[0, :N_NODES], deg_p[1, :N_NODES],
        W, b.reshape(1, D),
    )


# speedup vs baseline: 3.6582x; 1.0159x over previous
import functools

import jax
import jax.numpy as jnp
from jax import lax
from jax.experimental import pallas as pl
from jax.experimental.pallas import tpu as pltpu
from jax.experimental.pallas import tpu_sc as plsc

N_NODES = 10000
D = 128
NC = 2
NS = 16
CHUNK = 128
GRP = 8
N_PAD = 10112
BAND = N_PAD // NS
DEGW = 16


def _sc_aggregate(h, src, dst, groups_per_tile):
    mesh = plsc.VectorSubcoreMesh(core_axis_name="c", subcore_axis_name="s")

    @functools.partial(
        pl.kernel,
        out_type=[
            jax.ShapeDtypeStruct((NC, N_PAD, D), jnp.float32),
            jax.ShapeDtypeStruct((NC, N_PAD, DEGW), jnp.float32),
        ],
        mesh=mesh,
        compiler_params=pltpu.CompilerParams(use_tc_tiling_on_sc=False),
        scratch_types=[
            pltpu.VMEM((2, GRP, CHUNK), jnp.int32),
            pltpu.VMEM((2, GRP, CHUNK), jnp.int32),
            pltpu.VMEM((2, CHUNK, D), jnp.float32),
            pltpu.VMEM((CHUNK, DEGW), jnp.float32),
            pltpu.VMEM_SHARED((N_PAD, D), jnp.float32),
            pltpu.VMEM_SHARED((N_PAD, DEGW), jnp.float32),
            pltpu.SemaphoreType.DMA,
            pltpu.SemaphoreType.DMA,
            pltpu.SemaphoreType.DMA,
            pltpu.SemaphoreType.DMA,
            pltpu.SemaphoreType.DMA,
            pltpu.SemaphoreType.DMA,
        ],
    )
    def agg_kernel(h_hbm, src_hbm, dst_hbm, zagg_hbm, zdeg_hbm,
                   agg_out, deg_out,
                   src_v, dst_v, rows_v, ones_v,
                   agg_sh, deg_sh, isem, gsem0, gsem1, ssem0, ssem1, dsem):
        cid = lax.axis_index("c")
        sid = lax.axis_index("s")
        gsems = (gsem0, gsem1)
        ssems = (ssem0, ssem1)

        ones16 = jnp.ones((16,), jnp.float32)

        def fill_ones(i, _):
            ones_v[i] = ones16
            return 0

        lax.fori_loop(0, CHUNK, fill_ones, 0)

        pltpu.async_copy(src_hbm.at[cid, sid, pl.ds(0, GRP)], src_v.at[0], isem)
        pltpu.async_copy(dst_hbm.at[cid, sid, pl.ds(0, GRP)], dst_v.at[0], isem)

        base = sid * BAND
        za = pltpu.async_copy(zagg_hbm, agg_sh.at[pl.ds(base, BAND)], gsem0)
        zd = pltpu.async_copy(zdeg_hbm, deg_sh.at[pl.ds(base, BAND)], gsem1)
        za.wait()
        zd.wait()
        plsc.subcore_barrier()

        def group(g, _):
            cur = lax.rem(g, 2)
            sv = src_v.at[cur]
            dv = dst_v.at[cur]
            pltpu.make_async_copy(
                src_hbm.at[cid, sid, pl.ds(g * GRP, GRP)], sv, isem).wait()
            pltpu.make_async_copy(
                dst_hbm.at[cid, sid, pl.ds(g * GRP, GRP)], dv, isem).wait()

            @pl.when(g + 1 < groups_per_tile)
            def _prefetch():
                nxt = 1 - cur
                pltpu.async_copy(
                    src_hbm.at[cid, sid, pl.ds((g + 1) * GRP, GRP)],
                    src_v.at[nxt], isem)
                pltpu.async_copy(
                    dst_hbm.at[cid, sid, pl.ds((g + 1) * GRP, GRP)],
                    dst_v.at[nxt], isem)

            gd = {}
            sd = {}
            gd[0] = pltpu.async_copy(h_hbm.at[sv.at[0]], rows_v.at[0], gsem0)
            for j in range(GRP):
                b = j % 2
                gd[j].wait()
                if j + 1 < GRP:
                    if j >= 1:
                        sd[j - 1].wait()
                    gd[j + 1] = pltpu.async_copy(
                        h_hbm.at[sv.at[j + 1]], rows_v.at[1 - b], gsems[1 - b])
                sd[j] = pltpu.async_copy(
                    rows_v.at[b], agg_sh.at[dv.at[j]], ssems[b], add=True)
                pltpu.async_copy(ones_v, deg_sh.at[dv.at[j]], dsem, add=True)
            sd[GRP - 2].wait()
            sd[GRP - 1].wait()
            for j in range(GRP):
                pltpu.make_async_copy(
                    ones_v, deg_sh.at[dv.at[j]], dsem).wait()
            return 0

        lax.fori_loop(0, groups_per_tile, group, 0)
        plsc.subcore_barrier()

        pltpu.sync_copy(agg_sh.at[pl.ds(base, BAND)],
                        agg_out.at[cid, pl.ds(base, BAND)])
        pltpu.sync_copy(deg_sh.at[pl.ds(base, BAND)],
                        deg_out.at[cid, pl.ds(base, BAND)])

    zagg = jnp.zeros((BAND, D), jnp.float32)
    zdeg = jnp.zeros((BAND, DEGW), jnp.float32)
    return agg_kernel(h, src, dst, zagg, zdeg)


def _tc_body(h_ref, a0_ref, a1_ref, d0_ref, d1_ref, w_ref, b_ref, o_ref):
    deg = d0_ref[:, 0:1] + d1_ref[:, 0:1]
    agg = a0_ref[...] + a1_ref[...]
    mean = agg / jnp.maximum(deg, 1.0)
    h_new = jnp.where(deg > 0.0, mean, h_ref[...])
    acc = jnp.dot(h_new, w_ref[...], preferred_element_type=jnp.float32)
    o_ref[...] = jnp.maximum(acc + b_ref[...], 0.0)


def _tc_update(h, a0, a1, d0, d1, W, b):
    R = 2000
    grid = (N_NODES // R,)
    return pl.pallas_call(
        _tc_body,
        grid=grid,
        in_specs=[
            pl.BlockSpec((R, D), lambda i: (i, 0)),
            pl.BlockSpec((R, D), lambda i: (i, 0)),
            pl.BlockSpec((R, D), lambda i: (i, 0)),
            pl.BlockSpec((R, DEGW), lambda i: (i, 0)),
            pl.BlockSpec((R, DEGW), lambda i: (i, 0)),
            pl.BlockSpec((D, D), lambda i: (0, 0)),
            pl.BlockSpec((1, D), lambda i: (0, 0)),
        ],
        out_specs=pl.BlockSpec((R, D), lambda i: (i, 0)),
        out_shape=jax.ShapeDtypeStruct((N_NODES, D), jnp.float32),
    )(h, a0, a1, d0, d1, W, b)


def kernel(h, edge_index, W, b):
    src = edge_index[0].astype(jnp.int32)
    dst = edge_index[1].astype(jnp.int32)
    E = src.shape[0]
    lane = NC * NS * GRP * CHUNK
    groups_per_tile = -(-E // lane)
    e_pad = lane * groups_per_tile
    if e_pad != E:
        src = jnp.concatenate(
            [src, jnp.zeros((e_pad - E,), jnp.int32)])
        dst = jnp.concatenate(
            [dst, jnp.full((e_pad - E,), N_NODES, jnp.int32)])
    src = src.reshape(NC, NS, groups_per_tile * GRP, CHUNK)
    dst = dst.reshape(NC, NS, groups_per_tile * GRP, CHUNK)

    agg_p, deg_p = _sc_aggregate(h, src, dst, groups_per_tile)

    return _tc_update(
        h,
        agg_p[0, :N_NODES], agg_p[1, :N_NODES],
        deg_p[0, :N_NODES], deg_p[1, :N_NODES],
        W, b.reshape(1, D),
    )


# speedup vs baseline: 3.7837x; 1.0343x over previous
import functools

import jax
import jax.numpy as jnp
from jax import lax
from jax.experimental import pallas as pl
from jax.experimental.pallas import tpu as pltpu
from jax.experimental.pallas import tpu_sc as plsc

N_NODES = 10000
D = 128
NC = 2
NS = 16
CHUNK = 64
GRP = 16
NBUF = 4
N_PAD = 10112
BAND = N_PAD // NS
DEGW = 16


def _sc_aggregate(h, src, dst, groups_per_tile):
    mesh = plsc.VectorSubcoreMesh(core_axis_name="c", subcore_axis_name="s")

    @functools.partial(
        pl.kernel,
        out_type=[
            jax.ShapeDtypeStruct((NC, N_PAD, D), jnp.float32),
            jax.ShapeDtypeStruct((NC, N_PAD, DEGW), jnp.float32),
        ],
        mesh=mesh,
        compiler_params=pltpu.CompilerParams(use_tc_tiling_on_sc=False),
        scratch_types=[
            pltpu.VMEM((2, GRP, CHUNK), jnp.int32),
            pltpu.VMEM((2, GRP, CHUNK), jnp.int32),
            pltpu.VMEM((NBUF, CHUNK, D), jnp.float32),
            pltpu.VMEM((CHUNK, DEGW), jnp.float32),
            pltpu.VMEM_SHARED((N_PAD, D), jnp.float32),
            pltpu.VMEM_SHARED((N_PAD, DEGW), jnp.float32),
            pltpu.SemaphoreType.DMA,
            [pltpu.SemaphoreType.DMA] * NBUF,
            [pltpu.SemaphoreType.DMA] * NBUF,
            pltpu.SemaphoreType.DMA,
        ],
    )
    def agg_kernel(h_hbm, src_hbm, dst_hbm, zagg_hbm, zdeg_hbm,
                   agg_out, deg_out,
                   src_v, dst_v, rows_v, ones_v,
                   agg_sh, deg_sh, isem, gsems, ssems, dsem):
        cid = lax.axis_index("c")
        sid = lax.axis_index("s")

        ones16 = jnp.ones((16,), jnp.float32)

        def fill_ones(i, _):
            ones_v[i] = ones16
            return 0

        lax.fori_loop(0, CHUNK, fill_ones, 0)

        pltpu.async_copy(src_hbm.at[cid, sid, pl.ds(0, GRP)], src_v.at[0], isem)
        pltpu.async_copy(dst_hbm.at[cid, sid, pl.ds(0, GRP)], dst_v.at[0], isem)

        base = sid * BAND
        za = pltpu.async_copy(zagg_hbm, agg_sh.at[pl.ds(base, BAND)], gsems[0])
        zd = pltpu.async_copy(zdeg_hbm, deg_sh.at[pl.ds(base, BAND)], gsems[1])
        za.wait()
        zd.wait()
        plsc.subcore_barrier()

        def group(g, _):
            cur = lax.rem(g, 2)
            sv = src_v.at[cur]
            dv = dst_v.at[cur]
            pltpu.make_async_copy(
                src_hbm.at[cid, sid, pl.ds(g * GRP, GRP)], sv, isem).wait()
            pltpu.make_async_copy(
                dst_hbm.at[cid, sid, pl.ds(g * GRP, GRP)], dv, isem).wait()

            @pl.when(g + 1 < groups_per_tile)
            def _prefetch():
                nxt = 1 - cur
                pltpu.async_copy(
                    src_hbm.at[cid, sid, pl.ds((g + 1) * GRP, GRP)],
                    src_v.at[nxt], isem)
                pltpu.async_copy(
                    dst_hbm.at[cid, sid, pl.ds((g + 1) * GRP, GRP)],
                    dst_v.at[nxt], isem)

            gd = {}
            sd = {}
            for j in range(NBUF - 1):
                gd[j] = pltpu.async_copy(
                    h_hbm.at[sv.at[j]], rows_v.at[j], gsems[j])
            for j in range(GRP):
                b = j % NBUF
                jn = j + NBUF - 1
                if jn < GRP:
                    bn = jn % NBUF
                    if j >= 1:
                        sd[j - 1].wait()
                    gd[jn] = pltpu.async_copy(
                        h_hbm.at[sv.at[jn]], rows_v.at[bn], gsems[bn])
                gd[j].wait()
                sd[j] = pltpu.async_copy(
                    rows_v.at[b], agg_sh.at[dv.at[j]], ssems[b], add=True)
                pltpu.async_copy(ones_v, deg_sh.at[dv.at[j]], dsem, add=True)
            for j in range(GRP - NBUF, GRP):
                sd[j].wait()
            for j in range(GRP):
                pltpu.make_async_copy(
                    ones_v, deg_sh.at[dv.at[j]], dsem).wait()
            return 0

        lax.fori_loop(0, groups_per_tile, group, 0)
        plsc.subcore_barrier()

        pltpu.sync_copy(agg_sh.at[pl.ds(base, BAND)],
                        agg_out.at[cid, pl.ds(base, BAND)])
        pltpu.sync_copy(deg_sh.at[pl.ds(base, BAND)],
                        deg_out.at[cid, pl.ds(base, BAND)])

    zagg = jnp.zeros((BAND, D), jnp.float32)
    zdeg = jnp.zeros((BAND, DEGW), jnp.float32)
    return agg_kernel(h, src, dst, zagg, zdeg)


def _tc_body(h_ref, a0_ref, a1_ref, d0_ref, d1_ref, w_ref, b_ref, o_ref):
    deg = d0_ref[:, 0:1] + d1_ref[:, 0:1]
    agg = a0_ref[...] + a1_ref[...]
    mean = agg / jnp.maximum(deg, 1.0)
    h_new = jnp.where(deg > 0.0, mean, h_ref[...])
    acc = jnp.dot(h_new, w_ref[...], preferred_element_type=jnp.float32)
    o_ref[...] = jnp.maximum(acc + b_ref[...], 0.0)


def _tc_update(h, a0, a1, d0, d1, W, b):
    R = 2000
    grid = (N_NODES // R,)
    return pl.pallas_call(
        _tc_body,
        grid=grid,
        in_specs=[
            pl.BlockSpec((R, D), lambda i: (i, 0)),
            pl.BlockSpec((R, D), lambda i: (i, 0)),
            pl.BlockSpec((R, D), lambda i: (i, 0)),
            pl.BlockSpec((R, DEGW), lambda i: (i, 0)),
            pl.BlockSpec((R, DEGW), lambda i: (i, 0)),
            pl.BlockSpec((D, D), lambda i: (0, 0)),
            pl.BlockSpec((1, D), lambda i: (0, 0)),
        ],
        out_specs=pl.BlockSpec((R, D), lambda i: (i, 0)),
        out_shape=jax.ShapeDtypeStruct((N_NODES, D), jnp.float32),
    )(h, a0, a1, d0, d1, W, b)


def kernel(h, edge_index, W, b):
    src = edge_index[0].astype(jnp.int32)
    dst = edge_index[1].astype(jnp.int32)
    E = src.shape[0]
    lane = NC * NS * GRP * CHUNK
    groups_per_tile = -(-E // lane)
    e_pad = lane * groups_per_tile
    if e_pad != E:
        src = jnp.concatenate(
            [src, jnp.zeros((e_pad - E,), jnp.int32)])
        dst = jnp.concatenate(
            [dst, jnp.full((e_pad - E,), N_NODES, jnp.int32)])
    src = src.reshape(NC, NS, groups_per_tile * GRP, CHUNK)
    dst = dst.reshape(NC, NS, groups_per_tile * GRP, CHUNK)

    agg_p, deg_p = _sc_aggregate(h, src, dst, groups_per_tile)

    return _tc_update(
        h,
        agg_p[0, :N_NODES], agg_p[1, :N_NODES],
        deg_p[0, :N_NODES], deg_p[1, :N_NODES],
        W, b.reshape(1, D),
    )


# speedup vs baseline: 8.1713x; 2.1596x over previous
import functools

import jax
import jax.numpy as jnp
from jax import lax
from jax.experimental import pallas as pl
from jax.experimental.pallas import tpu as pltpu
from jax.experimental.pallas import tpu_sc as plsc

N_NODES = 10000
D = 128
DH = D // 2
NC = 2
NS = 16
CHUNK = 64
GRP = 16
NBUF = 8
N_PAD = 10112
BAND = N_PAD // NS
DEGW = 16


def _sc_aggregate(hsplit, src, dst, groups_per_tile):
    mesh = plsc.VectorSubcoreMesh(core_axis_name="c", subcore_axis_name="s")

    @functools.partial(
        pl.kernel,
        out_type=[
            jax.ShapeDtypeStruct((NC, N_PAD, DH), jnp.float32),
            jax.ShapeDtypeStruct((N_PAD, DEGW), jnp.float32),
        ],
        mesh=mesh,
        compiler_params=pltpu.CompilerParams(use_tc_tiling_on_sc=False),
        scratch_types=[
            pltpu.VMEM((2, GRP, CHUNK), jnp.int32),
            pltpu.VMEM((2, GRP, CHUNK), jnp.int32),
            pltpu.VMEM((NBUF, CHUNK, DH), jnp.float32),
            pltpu.VMEM((CHUNK, DEGW), jnp.float32),
            pltpu.VMEM_SHARED((N_PAD, DH), jnp.float32),
            pltpu.VMEM_SHARED((N_PAD, DH), jnp.float32),
            pltpu.VMEM_SHARED((N_PAD, DEGW), jnp.float32),
            pltpu.SemaphoreType.DMA,
            [pltpu.SemaphoreType.DMA] * NBUF,
            [pltpu.SemaphoreType.DMA] * NBUF,
            pltpu.SemaphoreType.DMA,
        ],
    )
    def agg_kernel(h_hbm, src_hbm, dst_hbm, zagg_hbm, zdeg_hbm,
                   agg_out, deg_out,
                   src_v, dst_v, rows_v, ones_v,
                   h_sh, agg_sh, deg_sh, isem, gsems, ssems, dsem):
        cid = lax.axis_index("c")
        sid = lax.axis_index("s")
        is_deg_core = cid == 0

        ones16 = jnp.ones((16,), jnp.float32)

        def fill_ones(i, _):
            ones_v[i] = ones16
            return 0

        lax.fori_loop(0, CHUNK, fill_ones, 0)

        pltpu.async_copy(src_hbm.at[sid, pl.ds(0, GRP)], src_v.at[0], isem)
        pltpu.async_copy(dst_hbm.at[sid, pl.ds(0, GRP)], dst_v.at[0], isem)

        base = sid * BAND
        hd = pltpu.async_copy(h_hbm.at[cid, pl.ds(base, BAND)],
                              h_sh.at[pl.ds(base, BAND)], gsems[0])
        za = pltpu.async_copy(zagg_hbm, agg_sh.at[pl.ds(base, BAND)], gsems[1])
        hd.wait()
        za.wait()

        @pl.when(is_deg_core)
        def _zero_deg():
            pltpu.async_copy(zdeg_hbm, deg_sh.at[pl.ds(base, BAND)],
                             gsems[2]).wait()

        plsc.subcore_barrier()

        def group(g, _):
            cur = lax.rem(g, 2)
            sv = src_v.at[cur]
            dv = dst_v.at[cur]
            pltpu.make_async_copy(
                src_hbm.at[sid, pl.ds(g * GRP, GRP)], sv, isem).wait()
            pltpu.make_async_copy(
                dst_hbm.at[sid, pl.ds(g * GRP, GRP)], dv, isem).wait()

            @pl.when(g + 1 < groups_per_tile)
            def _prefetch():
                nxt = 1 - cur
                pltpu.async_copy(
                    src_hbm.at[sid, pl.ds((g + 1) * GRP, GRP)],
                    src_v.at[nxt], isem)
                pltpu.async_copy(
                    dst_hbm.at[sid, pl.ds((g + 1) * GRP, GRP)],
                    dst_v.at[nxt], isem)

            @pl.when(is_deg_core)
            def _deg_scatter():
                for j in range(GRP):
                    pltpu.async_copy(ones_v, deg_sh.at[dv.at[j]], dsem,
                                     add=True)

            gd = {}
            sd = {}
            for j in range(NBUF - 1):
                gd[j] = pltpu.async_copy(
                    h_sh.at[sv.at[j]], rows_v.at[j], gsems[j])
            for j in range(GRP):
                b = j % NBUF
                jn = j + NBUF - 1
                if jn < GRP:
                    bn = jn % NBUF
                    if j >= 1:
                        sd[j - 1].wait()
                    gd[jn] = pltpu.async_copy(
                        h_sh.at[sv.at[jn]], rows_v.at[bn], gsems[bn])
                gd[j].wait()
                sd[j] = pltpu.async_copy(
                    rows_v.at[b], agg_sh.at[dv.at[j]], ssems[b], add=True)
            for j in range(GRP - NBUF, GRP):
                sd[j].wait()

            @pl.when(is_deg_core)
            def _deg_drain():
                for j in range(GRP):
                    pltpu.make_async_copy(
                        ones_v, deg_sh.at[dv.at[j]], dsem).wait()

            return 0

        lax.fori_loop(0, groups_per_tile, group, 0)
        plsc.subcore_barrier()

        pltpu.sync_copy(agg_sh.at[pl.ds(base, BAND)],
                        agg_out.at[cid, pl.ds(base, BAND)])

        @pl.when(is_deg_core)
        def _deg_out():
            pltpu.sync_copy(deg_sh.at[pl.ds(base, BAND)],
                            deg_out.at[pl.ds(base, BAND)])

    zagg = jnp.zeros((BAND, DH), jnp.float32)
    zdeg = jnp.zeros((BAND, DEGW), jnp.float32)
    return agg_kernel(hsplit, src, dst, zagg, zdeg)


def _tc_body(h_ref, a0_ref, a1_ref, d_ref, w_ref, b_ref, o_ref):
    deg = d_ref[:, 0:1]
    scale = 1.0 / jnp.maximum(deg, 1.0)
    gate = deg > 0.0
    hn0 = jnp.where(gate, a0_ref[...] * scale, h_ref[:, 0:DH])
    hn1 = jnp.where(gate, a1_ref[...] * scale, h_ref[:, DH:D])
    acc = (jnp.dot(hn0, w_ref[0:DH, :], preferred_element_type=jnp.float32)
           + jnp.dot(hn1, w_ref[DH:D, :], preferred_element_type=jnp.float32))
    o_ref[...] = jnp.maximum(acc + b_ref[...], 0.0)


def _tc_update(h, a0, a1, d, W, b):
    R = 2000
    grid = (N_NODES // R,)
    return pl.pallas_call(
        _tc_body,
        grid=grid,
        in_specs=[
            pl.BlockSpec((R, D), lambda i: (i, 0)),
            pl.BlockSpec((R, DH), lambda i: (i, 0)),
            pl.BlockSpec((R, DH), lambda i: (i, 0)),
            pl.BlockSpec((R, DEGW), lambda i: (i, 0)),
            pl.BlockSpec((D, D), lambda i: (0, 0)),
            pl.BlockSpec((1, D), lambda i: (0, 0)),
        ],
        out_specs=pl.BlockSpec((R, D), lambda i: (i, 0)),
        out_shape=jax.ShapeDtypeStruct((N_NODES, D), jnp.float32),
    )(h, a0, a1, d, W, b)


def kernel(h, edge_index, W, b):
    src = edge_index[0].astype(jnp.int32)
    dst = edge_index[1].astype(jnp.int32)
    E = src.shape[0]
    lane = NS * GRP * CHUNK
    groups_per_tile = -(-E // lane)
    e_pad = lane * groups_per_tile
    if e_pad != E:
        src = jnp.concatenate(
            [src, jnp.zeros((e_pad - E,), jnp.int32)])
        dst = jnp.concatenate(
            [dst, jnp.full((e_pad - E,), N_NODES, jnp.int32)])
    src = src.reshape(NS, groups_per_tile * GRP, CHUNK)
    dst = dst.reshape(NS, groups_per_tile * GRP, CHUNK)

    hp = jnp.concatenate(
        [h, jnp.zeros((N_PAD - N_NODES, D), jnp.float32)])
    hsplit = hp.reshape(N_PAD, NC, DH).transpose(1, 0, 2)

    agg_p, deg_p = _sc_aggregate(hsplit, src, dst, groups_per_tile)

    return _tc_update(
        h,
        agg_p[0, :N_NODES], agg_p[1, :N_NODES],
        deg_p[:N_NODES],
        W, b.reshape(1, D),
    )


# speedup vs baseline: 9.3596x; 1.1454x over previous
import functools

import jax
import jax.numpy as jnp
from jax import lax
from jax.experimental import pallas as pl
from jax.experimental.pallas import tpu as pltpu
from jax.experimental.pallas import tpu_sc as plsc

N_NODES = 10000
D = 128
DH = D // 2
NC = 2
NS = 16
CHUNK = 64
GRP = 16
NBUF = 8
N_PAD = 10112
BAND = N_PAD // NS
DEGW = 16


def _sc_aggregate(hsplit, src, dst, groups_per_tile):
    mesh = plsc.VectorSubcoreMesh(core_axis_name="c", subcore_axis_name="s")

    @functools.partial(
        pl.kernel,
        out_type=[
            jax.ShapeDtypeStruct((NC, N_PAD, DH), jnp.float32),
            jax.ShapeDtypeStruct((NC, N_PAD, DEGW), jnp.float32),
        ],
        mesh=mesh,
        compiler_params=pltpu.CompilerParams(use_tc_tiling_on_sc=False),
        scratch_types=[
            pltpu.VMEM((2, GRP, CHUNK), jnp.int32),
            pltpu.VMEM((2, GRP, CHUNK), jnp.int32),
            pltpu.VMEM((NBUF, CHUNK, DH), jnp.float32),
            pltpu.VMEM((CHUNK, DEGW), jnp.float32),
            pltpu.VMEM_SHARED((N_PAD, DH), jnp.float32),
            pltpu.VMEM_SHARED((N_PAD, DH), jnp.float32),
            pltpu.VMEM_SHARED((N_PAD, DEGW), jnp.float32),
            pltpu.SemaphoreType.DMA,
            [pltpu.SemaphoreType.DMA] * NBUF,
            [pltpu.SemaphoreType.DMA] * NBUF,
            pltpu.SemaphoreType.DMA,
        ],
    )
    def agg_kernel(h_hbm, src_hbm, dst_hbm, zagg_hbm, zdeg_hbm,
                   agg_out, deg_out,
                   src_v, dst_v, rows_v, ones_v,
                   h_sh, agg_sh, deg_sh, isem, gsems, ssems, dsem):
        cid = lax.axis_index("c")
        sid = lax.axis_index("s")
        deg_half = groups_per_tile // 2

        ones16 = jnp.ones((16,), jnp.float32)

        def fill_ones(i, _):
            ones_v[i] = ones16
            return 0

        lax.fori_loop(0, CHUNK, fill_ones, 0)

        pltpu.async_copy(src_hbm.at[sid, pl.ds(0, GRP)], src_v.at[0], isem)
        pltpu.async_copy(dst_hbm.at[sid, pl.ds(0, GRP)], dst_v.at[0], isem)

        base = sid * BAND
        col = cid * DH

        @pl.when(base + BAND <= N_NODES)
        def _stage_full():
            pltpu.async_copy(h_hbm.at[pl.ds(base, BAND), pl.ds(col, DH)],
                             h_sh.at[pl.ds(base, BAND)], gsems[0]).wait()

        @pl.when(base + BAND > N_NODES)
        def _stage_tail():
            tail = N_NODES - (NS - 1) * BAND
            pltpu.async_copy(
                h_hbm.at[pl.ds((NS - 1) * BAND, tail), pl.ds(col, DH)],
                h_sh.at[pl.ds((NS - 1) * BAND, tail)], gsems[0]).wait()

        za = pltpu.async_copy(zagg_hbm, agg_sh.at[pl.ds(base, BAND)], gsems[1])
        zd = pltpu.async_copy(zdeg_hbm, deg_sh.at[pl.ds(base, BAND)], gsems[2])
        za.wait()
        zd.wait()
        plsc.subcore_barrier()

        def group(g, _):
            cur = lax.rem(g, 2)
            sv = src_v.at[cur]
            dv = dst_v.at[cur]
            pltpu.make_async_copy(
                src_hbm.at[sid, pl.ds(g * GRP, GRP)], sv, isem).wait()
            pltpu.make_async_copy(
                dst_hbm.at[sid, pl.ds(g * GRP, GRP)], dv, isem).wait()

            @pl.when(g + 1 < groups_per_tile)
            def _prefetch():
                nxt = 1 - cur
                pltpu.async_copy(
                    src_hbm.at[sid, pl.ds((g + 1) * GRP, GRP)],
                    src_v.at[nxt], isem)
                pltpu.async_copy(
                    dst_hbm.at[sid, pl.ds((g + 1) * GRP, GRP)],
                    dst_v.at[nxt], isem)

            do_deg = lax.select(cid == 0, g < deg_half, g >= deg_half)

            @pl.when(do_deg)
            def _deg_scatter():
                for j in range(GRP):
                    pltpu.async_copy(ones_v, deg_sh.at[dv.at[j]], dsem,
                                     add=True)

            gd = {}
            sd = {}
            for j in range(NBUF - 1):
                gd[j] = pltpu.async_copy(
                    h_sh.at[sv.at[j]], rows_v.at[j], gsems[j])
            for j in range(GRP):
                b = j % NBUF
                jn = j + NBUF - 1
                if jn < GRP:
                    bn = jn % NBUF
                    if j >= 1:
                        sd[j - 1].wait()
                    gd[jn] = pltpu.async_copy(
                        h_sh.at[sv.at[jn]], rows_v.at[bn], gsems[bn])
                gd[j].wait()
                sd[j] = pltpu.async_copy(
                    rows_v.at[b], agg_sh.at[dv.at[j]], ssems[b], add=True)
            for j in range(GRP - NBUF, GRP):
                sd[j].wait()

            @pl.when(do_deg)
            def _deg_drain():
                for j in range(GRP):
                    pltpu.make_async_copy(
                        ones_v, deg_sh.at[dv.at[j]], dsem).wait()

            return 0

        lax.fori_loop(0, groups_per_tile, group, 0)
        plsc.subcore_barrier()

        pltpu.sync_copy(agg_sh.at[pl.ds(base, BAND)],
                        agg_out.at[cid, pl.ds(base, BAND)])
        pltpu.sync_copy(deg_sh.at[pl.ds(base, BAND)],
                        deg_out.at[cid, pl.ds(base, BAND)])

    zagg = jnp.zeros((BAND, DH), jnp.float32)
    zdeg = jnp.zeros((BAND, DEGW), jnp.float32)
    return agg_kernel(hsplit, src, dst, zagg, zdeg)


def _tc_body(h_ref, a0_ref, a1_ref, d0_ref, d1_ref, w_ref, b_ref, o_ref):
    deg = d0_ref[0, :, 0:1] + d1_ref[0, :, 0:1]
    scale = 1.0 / jnp.maximum(deg, 1.0)
    gate = deg > 0.0
    hn0 = jnp.where(gate, a0_ref[0] * scale, h_ref[:, 0:DH])
    hn1 = jnp.where(gate, a1_ref[0] * scale, h_ref[:, DH:D])
    acc = (jnp.dot(hn0, w_ref[0:DH, :], preferred_element_type=jnp.float32)
           + jnp.dot(hn1, w_ref[DH:D, :], preferred_element_type=jnp.float32))
    o_ref[...] = jnp.maximum(acc + b_ref[...], 0.0)


def _tc_update(h, agg, deg, W, b):
    R = 2000
    grid = (N_NODES // R,)
    return pl.pallas_call(
        _tc_body,
        grid=grid,
        in_specs=[
            pl.BlockSpec((R, D), lambda i: (i, 0)),
            pl.BlockSpec((1, R, DH), lambda i: (0, i, 0)),
            pl.BlockSpec((1, R, DH), lambda i: (1, i, 0)),
            pl.BlockSpec((1, R, DEGW), lambda i: (0, i, 0)),
            pl.BlockSpec((1, R, DEGW), lambda i: (1, i, 0)),
            pl.BlockSpec((D, D), lambda i: (0, 0)),
            pl.BlockSpec((1, D), lambda i: (0, 0)),
        ],
        out_specs=pl.BlockSpec((R, D), lambda i: (i, 0)),
        out_shape=jax.ShapeDtypeStruct((N_NODES, D), jnp.float32),
    )(h, agg, agg, deg, deg, W, b)


def kernel(h, edge_index, W, b):
    src = edge_index[0].astype(jnp.int32)
    dst = edge_index[1].astype(jnp.int32)
    E = src.shape[0]
    lane = NS * GRP * CHUNK
    groups_per_tile = -(-E // lane)
    e_pad = lane * groups_per_tile
    if e_pad != E:
        src = jnp.concatenate(
            [src, jnp.zeros((e_pad - E,), jnp.int32)])
        dst = jnp.concatenate(
            [dst, jnp.full((e_pad - E,), N_NODES, jnp.int32)])
    src = src.reshape(NS, groups_per_tile * GRP, CHUNK)
    dst = dst.reshape(NS, groups_per_tile * GRP, CHUNK)

    agg_p, deg_p = _sc_aggregate(h, src, dst, groups_per_tile)

    return _tc_update(h, agg_p, deg_p, W, b.reshape(1, D))
